# baseline (speedup 1.0000x reference)
import functools

import jax
import jax.numpy as jnp
from jax import lax
from jax.experimental import pallas as pl
from jax.experimental.pallas import tpu as pltpu
from jax.experimental.pallas import tpu_sc as plsc

_PAD = 100000
_D = 128
_KP = 56
_FEWP = 8
_BLK = 128
_SC_VB = 16
_SC_NVB = 4
_SC_GRP = _SC_NVB * _SC_VB


def _sc_gather(table, idx):
    n_tot = idx.shape[0]
    width = table.shape[1]
    info = plsc.get_sparse_core_info()
    nw = info.num_cores * info.num_subcores
    nc = info.num_cores
    b_per_w = n_tot // nw
    n_pairs = b_per_w // (2 * _SC_GRP)
    mesh = plsc.VectorSubcoreMesh(core_axis_name="c", subcore_axis_name="s")

    @functools.partial(
        pl.kernel,
        mesh=mesh,
        out_type=jax.ShapeDtypeStruct((n_tot, width), table.dtype),
        compiler_params=pltpu.CompilerParams(use_tc_tiling_on_sc=False),
        scratch_types=[
            pltpu.VMEM((b_per_w,), jnp.int32),
            pltpu.VMEM((2, _SC_GRP, width), table.dtype),
        ]
        + [pltpu.SemaphoreType.DMA] * (_SC_NVB + 2),
    )
    def gk(table_hbm, idx_hbm, out_hbm, idx_v, rows_v, *sems):
        gsems = sems[:_SC_NVB]
        wsems = sems[_SC_NVB:]
        wid = lax.axis_index("s") * nc + lax.axis_index("c")
        base = wid * b_per_w
        pltpu.sync_copy(idx_hbm.at[pl.ds(base, b_per_w)], idx_v)

        def pair(j, carry):
            for p in range(2):
                goff = (2 * j + p) * _SC_GRP

                @pl.when(j >= 1)
                def _drain():
                    pltpu.make_async_copy(
                        rows_v.at[p],
                        out_hbm.at[pl.ds(base, _SC_GRP)],
                        wsems[p]).wait()

                cps = []
                for q in range(_SC_NVB):
                    ivec = idx_v[pl.ds(goff + q * _SC_VB, _SC_VB)]
                    cps.append(pltpu.async_copy(
                        table_hbm.at[ivec],
                        rows_v.at[p, pl.ds(q * _SC_VB, _SC_VB)],
                        gsems[q]))
                for c in cps:
                    c.wait()
                pltpu.async_copy(
                    rows_v.at[p],
                    out_hbm.at[pl.ds(base + goff, _SC_GRP)],
                    wsems[p])
            return carry

        lax.fori_loop(0, n_pairs, pair, 0)
        for p in range(2):
            pltpu.make_async_copy(
                rows_v.at[p],
                out_hbm.at[pl.ds(base, _SC_GRP)],
                wsems[p]).wait()

    return gk(table, idx)


def _tc_body(refs, out):
    (qrl, qel, qrr, qer, qsl, qsr, qil, qir, qdl, qdr,
     srl, sel, srr, ser, ssl, ssr, sil, sir, sdl, sdr,
     w_r, w_e, wb, gb, g1t, g1b, glng, glnb, g2t, g2b, tmp,
     p1t, p1b, p2t, p2b, lng, lnb,
     wiht, whhth, whhtr, bih, bhh) = refs

    temp = jnp.clip(tmp[0, 0], 0.1, 5.0)

    def nenc(rel_e, ent_e, self_e, ids, deg, n):
        rel_e = rel_e.astype(jnp.float32)
        ent_e = ent_e.astype(jnp.float32)
        self_e = self_e.astype(jnp.float32)
        proj = (jnp.dot(rel_e.reshape(n * _KP, _D), w_r[...],
                        preferred_element_type=jnp.float32)
                + jnp.dot(ent_e.reshape(n * _KP, _D), w_e[...],
                          preferred_element_type=jnp.float32)
                + wb[...] + gb[...])
        proj = jnp.where(proj >= 0, proj, 0.01 * proj).reshape(n, _KP, _D)
        sc = jnp.sum(proj * self_e[:, None, :], axis=-1)
        sc = jnp.where(ids == _PAD, -1e9, sc)
        m = jnp.max(sc, axis=1, keepdims=True)
        e = jnp.exp(sc - m)
        attn = e / jnp.sum(e, axis=1, keepdims=True)
        agg = jnp.sum(attn[..., None] * proj, axis=1)
        h = jnp.dot(agg, g1t[...], preferred_element_type=jnp.float32) + g1b[...]
        mu = jnp.mean(h, axis=-1, keepdims=True)
        var = jnp.mean((h - mu) ** 2, axis=-1, keepdims=True)
        h = (h - mu) / jnp.sqrt(var + 1e-5) * glng[...] + glnb[...]
        h = jnp.maximum(h, 0.0)
        gl = jnp.dot(h, g2t[...], preferred_element_type=jnp.float32) + g2b[...]
        gate = jax.nn.sigmoid(gl / temp)
        gate = gate * (deg > 0).astype(jnp.float32)
        return jnp.tanh(self_e + gate * agg)

    def senc(x):
        o = jnp.maximum(
            jnp.dot(x, p1t[...], preferred_element_type=jnp.float32) + p1b[...],
            0.0)
        o = jnp.dot(o, p2t[...], preferred_element_type=jnp.float32) + p2b[...]
        o = o + x
        mu = jnp.mean(o, axis=-1, keepdims=True)
        var = jnp.mean((o - mu) ** 2, axis=-1, keepdims=True)
        return (o - mu) / jnp.sqrt(var + 1e-5) * lng[...] + lnb[...]

    s_left = nenc(srl[...], sel[...], ssl[...], sil[...], sdl[...], _FEWP)
    s_right = nenc(srr[...], ser[...], ssr[...], sir[...], sdr[...], _FEWP)
    s_enc = senc(jnp.concatenate([s_left, s_right], axis=-1))
    rowmask = (lax.broadcasted_iota(jnp.int32, (_FEWP, 1), 0) < 5
               ).astype(jnp.float32)
    support_g = jnp.sum(s_enc * rowmask, axis=0, keepdims=True) / 5.0

    q_left = nenc(qrl[...], qel[...], qsl[...], qil[...], qdl[...], _BLK)
    q_right = nenc(qrr[...], qer[...], qsr[...], qir[...], qdr[...], _BLK)
    q_enc = senc(jnp.concatenate([q_left, q_right], axis=-1))

    rcon = jnp.dot(support_g, whhtr[...], preferred_element_type=jnp.float32)
    qw = (jnp.dot(q_enc, wiht[...], preferred_element_type=jnp.float32)
          + bih[...] + bhh[...])
    c = jnp.zeros((_BLK, 512), jnp.float32)
    h = None
    for step in range(4):
        if step == 0:
            gates = qw
        else:
            gates = qw + jnp.dot(h, whhth[...],
                                 preferred_element_type=jnp.float32) + rcon
        gi = gates[:, 0:512]
        gf = gates[:, 512:1024]
        gg = gates[:, 1024:1536]
        go = gates[:, 1536:2048]
        c = jax.nn.sigmoid(gf) * c + jax.nn.sigmoid(gi) * jnp.tanh(gg)
        h = q_enc + (jax.nn.sigmoid(go) * jnp.tanh(c))[:, 0:256]

    out[...] = jnp.sum(h * support_g, axis=-1, keepdims=True)


def _tc_forward(qrl, qel, qrr, qer, qsl, qsr, qil, qir, qdl, qdr,
                srl, sel, srr, ser, ssl, ssr, sil, sir, sdl, sdr,
                w_r, w_e, wb, gb, g1t, g1b, glng, glnb, g2t, g2b, tmp,
                p1t, p1b, p2t, p2b, lng, lnb,
                wiht, whhth, whhtr, bih, bhh):
    b = qrl.shape[0]
    grid = b // _BLK

    def bi3(i):
        return (i, 0, 0)

    def bi2(i):
        return (i, 0)

    def c3(i):
        return (0, 0, 0)

    def c2(i):
        return (0, 0)

    in_specs = [
        pl.BlockSpec((_BLK, _KP, _D), bi3),
        pl.BlockSpec((_BLK, _KP, _D), bi3),
        pl.BlockSpec((_BLK, _KP, _D), bi3),
        pl.BlockSpec((_BLK, _KP, _D), bi3),
        pl.BlockSpec((_BLK, _D), bi2),
        pl.BlockSpec((_BLK, _D), bi2),
        pl.BlockSpec((_BLK, _KP), bi2),
        pl.BlockSpec((_BLK, _KP), bi2),
        pl.BlockSpec((_BLK, 1), bi2),
        pl.BlockSpec((_BLK, 1), bi2),
        pl.BlockSpec((_FEWP, _KP, _D), c3),
        pl.BlockSpec((_FEWP, _KP, _D), c3),
        pl.BlockSpec((_FEWP, _KP, _D), c3),
        pl.BlockSpec((_FEWP, _KP, _D), c3),
        pl.BlockSpec((_FEWP, _D), c2),
        pl.BlockSpec((_FEWP, _D), c2),
        pl.BlockSpec((_FEWP, _KP), c2),
        pl.BlockSpec((_FEWP, _KP), c2),
        pl.BlockSpec((_FEWP, 1), c2),
        pl.BlockSpec((_FEWP, 1), c2),
        pl.BlockSpec((_D, _D), c2),
        pl.BlockSpec((_D, _D), c2),
        pl.BlockSpec((1, _D), c2),
        pl.BlockSpec((1, _D), c2),
        pl.BlockSpec((_D, 64), c2),
        pl.BlockSpec((1, 64), c2),
        pl.BlockSpec((1, 64), c2),
        pl.BlockSpec((1, 64), c2),
        pl.BlockSpec((64, 1), c2),
        pl.BlockSpec((1, 1), c2),
        pl.BlockSpec((1, 1), c2),
        pl.BlockSpec((256, 512), c2),
        pl.BlockSpec((1, 512), c2),
        pl.BlockSpec((512, 256), c2),
        pl.BlockSpec((1, 256), c2),
        pl.BlockSpec((1, 256), c2),
        pl.BlockSpec((1, 256), c2),
        pl.BlockSpec((256, 2048), c2),
        pl.BlockSpec((256, 2048), c2),
        pl.BlockSpec((256, 2048), c2),
        pl.BlockSpec((1, 2048), c2),
        pl.BlockSpec((1, 2048), c2),
    ]

    def body(*args):
        _tc_body(args[:-1], args[-1])

    return pl.pallas_call(
        body,
        grid=(grid,),
        in_specs=in_specs,
        out_specs=pl.BlockSpec((_BLK, 1), bi2),
        out_shape=jax.ShapeDtypeStruct((b, 1), jnp.float32),
    )(qrl, qel, qrr, qer, qsl, qsr, qil, qir, qdl, qdr,
      srl, sel, srr, ser, ssl, ssr, sil, sir, sdl, sdr,
      w_r, w_e, wb, gb, g1t, g1b, glng, glnb, g2t, g2b, tmp,
      p1t, p1b, p2t, p2b, lng, lnb,
      wiht, whhth, whhtr, bih, bhh)


def _pad_cols(ids):
    n, k = ids.shape
    return jnp.concatenate(
        [ids, jnp.full((n, _KP - k), _PAD, ids.dtype)], axis=1)


def kernel(query, support, q_l1, q_deg_l, q_r1, q_deg_r, s_l1, s_deg_l,
           s_r1, s_deg_r, symbol_emb, gcn_w_W, gcn_w_b, gcn_b, gate_temp,
           g1_W, g1_b, gln_g, gln_b, g2_W, g2_b, se_p1_W, se_p1_b, se_p2_W,
           se_p2_b, se_ln_g, se_ln_b, lstm_Wih, lstm_Whh, lstm_bih, lstm_bhh):
    b = query.shape[0]
    few = support.shape[0]

    pr = _FEWP - few
    s_l1p = jnp.concatenate(
        [s_l1, jnp.full((pr,) + s_l1.shape[1:], _PAD, s_l1.dtype)], axis=0)
    s_r1p = jnp.concatenate(
        [s_r1, jnp.full((pr,) + s_r1.shape[1:], _PAD, s_r1.dtype)], axis=0)
    supp = jnp.concatenate(
        [support, jnp.full((pr, 2), _PAD, support.dtype)], axis=0)
    sdl = jnp.concatenate([s_deg_l, jnp.zeros((pr,), s_deg_l.dtype)])
    sdr = jnp.concatenate([s_deg_r, jnp.zeros((pr,), s_deg_r.dtype)])

    qil = _pad_cols(q_l1[:, :, 0])
    qel_i = _pad_cols(q_l1[:, :, 1])
    qir = _pad_cols(q_r1[:, :, 0])
    qer_i = _pad_cols(q_r1[:, :, 1])
    sil = _pad_cols(s_l1p[:, :, 0])
    sel_i = _pad_cols(s_l1p[:, :, 1])
    sir = _pad_cols(s_r1p[:, :, 0])
    ser_i = _pad_cols(s_r1p[:, :, 1])

    nq = b * _KP
    ns = _FEWP * _KP
    segs = [qil, qel_i, qir, qer_i,
            query[:, 0], query[:, 1],
            sil, sel_i, sir, ser_i,
            supp[:, 0], supp[:, 1]]
    idx = jnp.concatenate([s.reshape(-1) for s in segs]).astype(jnp.int32)
    n_used = idx.shape[0]
    align = 32 * 2 * _SC_GRP
    n_tot = ((n_used + align - 1) // align) * align
    idx = jnp.concatenate(
        [idx, jnp.full((n_tot - n_used,), _PAD, jnp.int32)])

    table_bf = symbol_emb.astype(jnp.bfloat16)
    table_i32 = lax.bitcast_convert_type(
        table_bf.reshape(table_bf.shape[0], _D // 2, 2), jnp.int32)
    rows_i32 = _sc_gather(table_i32, idx)
    rows = lax.bitcast_convert_type(
        rows_i32, jnp.bfloat16).reshape(-1, _D)

    offs = []
    o = 0
    for s in segs:
        offs.append(o)
        o += s.size
    qrl = rows[offs[0]:offs[0] + nq].reshape(b, _KP, _D)
    qel = rows[offs[1]:offs[1] + nq].reshape(b, _KP, _D)
    qrr = rows[offs[2]:offs[2] + nq].reshape(b, _KP, _D)
    qer = rows[offs[3]:offs[3] + nq].reshape(b, _KP, _D)
    qsl = rows[offs[4]:offs[4] + b]
    qsr = rows[offs[5]:offs[5] + b]
    srl = rows[offs[6]:offs[6] + ns].reshape(_FEWP, _KP, _D)
    sel = rows[offs[7]:offs[7] + ns].reshape(_FEWP, _KP, _D)
    srr = rows[offs[8]:offs[8] + ns].reshape(_FEWP, _KP, _D)
    ser = rows[offs[9]:offs[9] + ns].reshape(_FEWP, _KP, _D)
    ssl = rows[offs[10]:offs[10] + _FEWP]
    ssr = rows[offs[11]:offs[11] + _FEWP]

    wt = gcn_w_W.T
    w_r, w_e = wt[:_D], wt[_D:]
    whht = lstm_Whh.T

    out = _tc_forward(
        qrl, qel, qrr, qer, qsl, qsr, qil, qir,
        q_deg_l.reshape(b, 1), q_deg_r.reshape(b, 1),
        srl, sel, srr, ser, ssl, ssr, sil, sir,
        sdl.reshape(_FEWP, 1), sdr.reshape(_FEWP, 1),
        w_r, w_e, gcn_w_b.reshape(1, _D), gcn_b.reshape(1, _D),
        g1_W.T, g1_b.reshape(1, 64), gln_g.reshape(1, 64),
        gln_b.reshape(1, 64), g2_W.T, g2_b.reshape(1, 1),
        gate_temp.reshape(1, 1),
        se_p1_W.T, se_p1_b.reshape(1, 512), se_p2_W.T,
        se_p2_b.reshape(1, 256), se_ln_g.reshape(1, 256),
        se_ln_b.reshape(1, 256),
        lstm_Wih.T, whht[:256], whht[256:],
        lstm_bih.reshape(1, 2048), lstm_bhh.reshape(1, 2048))
    return out.reshape(b)


# speedup vs baseline: 1.1308x; 1.1308x over previous
import functools

import jax
import jax.numpy as jnp
from jax import lax
from jax.experimental import pallas as pl
from jax.experimental.pallas import tpu as pltpu
from jax.experimental.pallas import tpu_sc as plsc

_PAD = 100000
_D = 128
_KP = 56
_FEWP = 8
_BLK = 128
_SC_VB = 16
_SC_NVB = 4
_SC_GRP = _SC_NVB * _SC_VB


def _sc_gather(table, idx):
    n_tot = idx.shape[0]
    width = table.shape[1]
    info = plsc.get_sparse_core_info()
    nw = info.num_cores * info.num_subcores
    nc = info.num_cores
    b_per_w = n_tot // nw
    n_pairs = b_per_w // (2 * _SC_GRP)
    mesh = plsc.VectorSubcoreMesh(core_axis_name="c", subcore_axis_name="s")

    @functools.partial(
        pl.kernel,
        mesh=mesh,
        out_type=jax.ShapeDtypeStruct((n_tot, width), table.dtype),
        compiler_params=pltpu.CompilerParams(use_tc_tiling_on_sc=True),
        scratch_types=[
            pltpu.VMEM((b_per_w,), jnp.int32),
            pltpu.VMEM((2, _SC_GRP, width), table.dtype),
        ]
        + [pltpu.SemaphoreType.DMA] * (_SC_NVB + 2),
    )
    def gk(table_hbm, idx_hbm, out_hbm, idx_v, rows_v, *sems):
        gsems = sems[:_SC_NVB]
        wsems = sems[_SC_NVB:]
        wid = lax.axis_index("s") * nc + lax.axis_index("c")
        base = wid * b_per_w
        pltpu.sync_copy(idx_hbm.at[pl.ds(base, b_per_w)], idx_v)

        def pair(j, carry):
            for p in range(2):
                goff = (2 * j + p) * _SC_GRP

                @pl.when(j >= 1)
                def _drain():
                    pltpu.make_async_copy(
                        rows_v.at[p],
                        out_hbm.at[pl.ds(base, _SC_GRP)],
                        wsems[p]).wait()

                cps = []
                for q in range(_SC_NVB):
                    ivec = idx_v[pl.ds(goff + q * _SC_VB, _SC_VB)]
                    cps.append(pltpu.async_copy(
                        table_hbm.at[ivec],
                        rows_v.at[p, pl.ds(q * _SC_VB, _SC_VB)],
                        gsems[q]))
                for c in cps:
                    c.wait()
                pltpu.async_copy(
                    rows_v.at[p],
                    out_hbm.at[pl.ds(base + goff, _SC_GRP)],
                    wsems[p])
            return carry

        lax.fori_loop(0, n_pairs, pair, 0)
        for p in range(2):
            pltpu.make_async_copy(
                rows_v.at[p],
                out_hbm.at[pl.ds(base, _SC_GRP)],
                wsems[p]).wait()

    return gk(table, idx)


def _tc_body(refs, out):
    (qrl, qel, qrr, qer, qsl, qsr, qil, qir, qdl, qdr,
     srl, sel, srr, ser, ssl, ssr, sil, sir, sdl, sdr,
     w_r, w_e, wb, gb, g1t, g1b, glng, glnb, g2t, g2b, tmp,
     p1t, p1b, p2t, p2b, lng, lnb,
     wiht, whhth, whhtr, bih, bhh) = refs

    temp = jnp.clip(tmp[0, 0], 0.1, 5.0)

    def nenc(rel_e, ent_e, self_e, ids, deg, n):
        rel_e = rel_e.astype(jnp.float32)
        ent_e = ent_e.astype(jnp.float32)
        self_e = self_e.astype(jnp.float32)
        proj = (jnp.dot(rel_e.reshape(n * _KP, _D), w_r[...],
                        preferred_element_type=jnp.float32)
                + jnp.dot(ent_e.reshape(n * _KP, _D), w_e[...],
                          preferred_element_type=jnp.float32)
                + wb[...] + gb[...])
        proj = jnp.where(proj >= 0, proj, 0.01 * proj).reshape(n, _KP, _D)
        sc = jnp.sum(proj * self_e[:, None, :], axis=-1)
        sc = jnp.where(ids == _PAD, -1e9, sc)
        m = jnp.max(sc, axis=1, keepdims=True)
        e = jnp.exp(sc - m)
        attn = e / jnp.sum(e, axis=1, keepdims=True)
        agg = jnp.sum(attn[..., None] * proj, axis=1)
        h = jnp.dot(agg, g1t[...], preferred_element_type=jnp.float32) + g1b[...]
        mu = jnp.mean(h, axis=-1, keepdims=True)
        var = jnp.mean((h - mu) ** 2, axis=-1, keepdims=True)
        h = (h - mu) / jnp.sqrt(var + 1e-5) * glng[...] + glnb[...]
        h = jnp.maximum(h, 0.0)
        gl = jnp.dot(h, g2t[...], preferred_element_type=jnp.float32) + g2b[...]
        gate = jax.nn.sigmoid(gl / temp)
        gate = gate * (deg > 0).astype(jnp.float32)
        return jnp.tanh(self_e + gate * agg)

    def senc(x):
        o = jnp.maximum(
            jnp.dot(x, p1t[...], preferred_element_type=jnp.float32) + p1b[...],
            0.0)
        o = jnp.dot(o, p2t[...], preferred_element_type=jnp.float32) + p2b[...]
        o = o + x
        mu = jnp.mean(o, axis=-1, keepdims=True)
        var = jnp.mean((o - mu) ** 2, axis=-1, keepdims=True)
        return (o - mu) / jnp.sqrt(var + 1e-5) * lng[...] + lnb[...]

    s_left = nenc(srl[...], sel[...], ssl[...], sil[...], sdl[...], _FEWP)
    s_right = nenc(srr[...], ser[...], ssr[...], sir[...], sdr[...], _FEWP)
    s_enc = senc(jnp.concatenate([s_left, s_right], axis=-1))
    rowmask = (lax.broadcasted_iota(jnp.int32, (_FEWP, 1), 0) < 5
               ).astype(jnp.float32)
    support_g = jnp.sum(s_enc * rowmask, axis=0, keepdims=True) / 5.0

    q_left = nenc(qrl[...], qel[...], qsl[...], qil[...], qdl[...], _BLK)
    q_right = nenc(qrr[...], qer[...], qsr[...], qir[...], qdr[...], _BLK)
    q_enc = senc(jnp.concatenate([q_left, q_right], axis=-1))

    rcon = jnp.dot(support_g, whhtr[...], preferred_element_type=jnp.float32)
    qw = (jnp.dot(q_enc, wiht[...], preferred_element_type=jnp.float32)
          + bih[...] + bhh[...])
    c = jnp.zeros((_BLK, 512), jnp.float32)
    h = None
    for step in range(4):
        if step == 0:
            gates = qw
        else:
            gates = qw + jnp.dot(h, whhth[...],
                                 preferred_element_type=jnp.float32) + rcon
        gi = gates[:, 0:512]
        gf = gates[:, 512:1024]
        gg = gates[:, 1024:1536]
        go = gates[:, 1536:2048]
        c = jax.nn.sigmoid(gf) * c + jax.nn.sigmoid(gi) * jnp.tanh(gg)
        h = q_enc + (jax.nn.sigmoid(go) * jnp.tanh(c))[:, 0:256]

    out[...] = jnp.sum(h * support_g, axis=-1, keepdims=True)


def _tc_forward(qrl, qel, qrr, qer, qsl, qsr, qil, qir, qdl, qdr,
                srl, sel, srr, ser, ssl, ssr, sil, sir, sdl, sdr,
                w_r, w_e, wb, gb, g1t, g1b, glng, glnb, g2t, g2b, tmp,
                p1t, p1b, p2t, p2b, lng, lnb,
                wiht, whhth, whhtr, bih, bhh):
    b = qrl.shape[0]
    grid = b // _BLK

    def bi3(i):
        return (i, 0, 0)

    def bi2(i):
        return (i, 0)

    def c3(i):
        return (0, 0, 0)

    def c2(i):
        return (0, 0)

    in_specs = [
        pl.BlockSpec((_BLK, _KP, _D), bi3),
        pl.BlockSpec((_BLK, _KP, _D), bi3),
        pl.BlockSpec((_BLK, _KP, _D), bi3),
        pl.BlockSpec((_BLK, _KP, _D), bi3),
        pl.BlockSpec((_BLK, _D), bi2),
        pl.BlockSpec((_BLK, _D), bi2),
        pl.BlockSpec((_BLK, _KP), bi2),
        pl.BlockSpec((_BLK, _KP), bi2),
        pl.BlockSpec((_BLK, 1), bi2),
        pl.BlockSpec((_BLK, 1), bi2),
        pl.BlockSpec((_FEWP, _KP, _D), c3),
        pl.BlockSpec((_FEWP, _KP, _D), c3),
        pl.BlockSpec((_FEWP, _KP, _D), c3),
        pl.BlockSpec((_FEWP, _KP, _D), c3),
        pl.BlockSpec((_FEWP, _D), c2),
        pl.BlockSpec((_FEWP, _D), c2),
        pl.BlockSpec((_FEWP, _KP), c2),
        pl.BlockSpec((_FEWP, _KP), c2),
        pl.BlockSpec((_FEWP, 1), c2),
        pl.BlockSpec((_FEWP, 1), c2),
        pl.BlockSpec((_D, _D), c2),
        pl.BlockSpec((_D, _D), c2),
        pl.BlockSpec((1, _D), c2),
        pl.BlockSpec((1, _D), c2),
        pl.BlockSpec((_D, 64), c2),
        pl.BlockSpec((1, 64), c2),
        pl.BlockSpec((1, 64), c2),
        pl.BlockSpec((1, 64), c2),
        pl.BlockSpec((64, 1), c2),
        pl.BlockSpec((1, 1), c2),
        pl.BlockSpec((1, 1), c2),
        pl.BlockSpec((256, 512), c2),
        pl.BlockSpec((1, 512), c2),
        pl.BlockSpec((512, 256), c2),
        pl.BlockSpec((1, 256), c2),
        pl.BlockSpec((1, 256), c2),
        pl.BlockSpec((1, 256), c2),
        pl.BlockSpec((256, 2048), c2),
        pl.BlockSpec((256, 2048), c2),
        pl.BlockSpec((256, 2048), c2),
        pl.BlockSpec((1, 2048), c2),
        pl.BlockSpec((1, 2048), c2),
    ]

    def body(*args):
        _tc_body(args[:-1], args[-1])

    return pl.pallas_call(
        body,
        grid=(grid,),
        in_specs=in_specs,
        out_specs=pl.BlockSpec((_BLK, 1), bi2),
        out_shape=jax.ShapeDtypeStruct((b, 1), jnp.float32),
    )(qrl, qel, qrr, qer, qsl, qsr, qil, qir, qdl, qdr,
      srl, sel, srr, ser, ssl, ssr, sil, sir, sdl, sdr,
      w_r, w_e, wb, gb, g1t, g1b, glng, glnb, g2t, g2b, tmp,
      p1t, p1b, p2t, p2b, lng, lnb,
      wiht, whhth, whhtr, bih, bhh)


def _pad_cols(ids):
    n, k = ids.shape
    return jnp.concatenate(
        [ids, jnp.full((n, _KP - k), _PAD, ids.dtype)], axis=1)


def kernel(query, support, q_l1, q_deg_l, q_r1, q_deg_r, s_l1, s_deg_l,
           s_r1, s_deg_r, symbol_emb, gcn_w_W, gcn_w_b, gcn_b, gate_temp,
           g1_W, g1_b, gln_g, gln_b, g2_W, g2_b, se_p1_W, se_p1_b, se_p2_W,
           se_p2_b, se_ln_g, se_ln_b, lstm_Wih, lstm_Whh, lstm_bih, lstm_bhh):
    b = query.shape[0]
    few = support.shape[0]

    pr = _FEWP - few
    s_l1p = jnp.concatenate(
        [s_l1, jnp.full((pr,) + s_l1.shape[1:], _PAD, s_l1.dtype)], axis=0)
    s_r1p = jnp.concatenate(
        [s_r1, jnp.full((pr,) + s_r1.shape[1:], _PAD, s_r1.dtype)], axis=0)
    supp = jnp.concatenate(
        [support, jnp.full((pr, 2), _PAD, support.dtype)], axis=0)
    sdl = jnp.concatenate([s_deg_l, jnp.zeros((pr,), s_deg_l.dtype)])
    sdr = jnp.concatenate([s_deg_r, jnp.zeros((pr,), s_deg_r.dtype)])

    qil = _pad_cols(q_l1[:, :, 0])
    qel_i = _pad_cols(q_l1[:, :, 1])
    qir = _pad_cols(q_r1[:, :, 0])
    qer_i = _pad_cols(q_r1[:, :, 1])
    sil = _pad_cols(s_l1p[:, :, 0])
    sel_i = _pad_cols(s_l1p[:, :, 1])
    sir = _pad_cols(s_r1p[:, :, 0])
    ser_i = _pad_cols(s_r1p[:, :, 1])

    nq = b * _KP
    ns = _FEWP * _KP
    segs = [qil, qel_i, qir, qer_i,
            query[:, 0], query[:, 1],
            sil, sel_i, sir, ser_i,
            supp[:, 0], supp[:, 1]]
    idx = jnp.concatenate([s.reshape(-1) for s in segs]).astype(jnp.int32)
    n_used = idx.shape[0]
    align = 32 * 2 * _SC_GRP
    n_tot = ((n_used + align - 1) // align) * align
    idx = jnp.concatenate(
        [idx, jnp.full((n_tot - n_used,), _PAD, jnp.int32)])

    rows = _sc_gather(symbol_emb, idx)

    offs = []
    o = 0
    for s in segs:
        offs.append(o)
        o += s.size
    qrl = rows[offs[0]:offs[0] + nq].reshape(b, _KP, _D)
    qel = rows[offs[1]:offs[1] + nq].reshape(b, _KP, _D)
    qrr = rows[offs[2]:offs[2] + nq].reshape(b, _KP, _D)
    qer = rows[offs[3]:offs[3] + nq].reshape(b, _KP, _D)
    qsl = rows[offs[4]:offs[4] + b]
    qsr = rows[offs[5]:offs[5] + b]
    srl = rows[offs[6]:offs[6] + ns].reshape(_FEWP, _KP, _D)
    sel = rows[offs[7]:offs[7] + ns].reshape(_FEWP, _KP, _D)
    srr = rows[offs[8]:offs[8] + ns].reshape(_FEWP, _KP, _D)
    ser = rows[offs[9]:offs[9] + ns].reshape(_FEWP, _KP, _D)
    ssl = rows[offs[10]:offs[10] + _FEWP]
    ssr = rows[offs[11]:offs[11] + _FEWP]

    wt = gcn_w_W.T
    w_r, w_e = wt[:_D], wt[_D:]
    whht = lstm_Whh.T

    out = _tc_forward(
        qrl, qel, qrr, qer, qsl, qsr, qil, qir,
        q_deg_l.reshape(b, 1), q_deg_r.reshape(b, 1),
        srl, sel, srr, ser, ssl, ssr, sil, sir,
        sdl.reshape(_FEWP, 1), sdr.reshape(_FEWP, 1),
        w_r, w_e, gcn_w_b.reshape(1, _D), gcn_b.reshape(1, _D),
        g1_W.T, g1_b.reshape(1, 64), gln_g.reshape(1, 64),
        gln_b.reshape(1, 64), g2_W.T, g2_b.reshape(1, 1),
        gate_temp.reshape(1, 1),
        se_p1_W.T, se_p1_b.reshape(1, 512), se_p2_W.T,
        se_p2_b.reshape(1, 256), se_ln_g.reshape(1, 256),
        se_ln_b.reshape(1, 256),
        lstm_Wih.T, whht[:256], whht[256:],
        lstm_bih.reshape(1, 2048), lstm_bhh.reshape(1, 2048))
    return out.reshape(b)
